# baseline (device time: 29855 ns/iter reference)
import jax
import jax.numpy as jnp
from jax import lax
from jax.experimental import pallas as pl
from jax.experimental.pallas import tpu as pltpu

N_DEV = 32
B, SQ, D = 2, 128, 512
HQ_SHARD = 8
GQA = 4
DH = 64
ROWS = B * SQ
CHUNK = ROWS // N_DEV


def _allreduce_body(p_ref, out_ref, sendbuf, recvbuf, send_sems, recv_sems,
                    credit_sem):
    me = lax.axis_index("i")
    left = lax.rem(me + N_DEV - 1, N_DEV)
    right = lax.rem(me + 1, N_DEV)

    barrier = pltpu.get_barrier_semaphore()
    for nbr in (left, right):
        pl.semaphore_signal(barrier, inc=1, device_id=(nbr,),
                            device_id_type=pl.DeviceIdType.MESH)
    pl.semaphore_wait(barrier, 2)

    out_ref[...] = p_ref[...]

    total_steps = 2 * (N_DEV - 1)
    for h in range(total_steps):
        slot = h % 2
        rs_phase = h < N_DEV - 1
        if rs_phase:
            c_send = lax.rem(me - h + 2 * N_DEV, N_DEV)
            c_recv = lax.rem(me - h - 1 + 2 * N_DEV, N_DEV)
        else:
            g = h - (N_DEV - 1)
            c_send = lax.rem(me + 1 - g + 2 * N_DEV, N_DEV)
            c_recv = lax.rem(me - g + 2 * N_DEV, N_DEV)

        sendbuf[slot] = out_ref[pl.ds(c_send * CHUNK, CHUNK), :]

        if h >= 2:
            pl.semaphore_wait(credit_sem, 1)

        rdma = pltpu.make_async_remote_copy(
            src_ref=sendbuf.at[slot],
            dst_ref=recvbuf.at[slot],
            send_sem=send_sems.at[slot],
            recv_sem=recv_sems.at[slot],
            device_id=(right,),
            device_id_type=pl.DeviceIdType.MESH,
        )
        rdma.start()
        rdma.wait()

        if rs_phase:
            cur = out_ref[pl.ds(c_recv * CHUNK, CHUNK), :]
            out_ref[pl.ds(c_recv * CHUNK, CHUNK), :] = cur + recvbuf[slot]
        else:
            out_ref[pl.ds(c_recv * CHUNK, CHUNK), :] = recvbuf[slot]

        if h < total_steps - 2:
            pl.semaphore_signal(credit_sem, inc=1, device_id=(left,),
                                device_id_type=pl.DeviceIdType.MESH)


_PLANE = 8
_NZ = 4
_BLK1 = ROWS // _PLANE
_BLK2 = _BLK1 // _NZ


def _entry_barrier(z, q):
    barrier = pltpu.get_barrier_semaphore()
    n_peers = 0
    for dq in range(1, _PLANE):
        tgt = z * _PLANE + lax.rem(q + dq, _PLANE)
        pl.semaphore_signal(barrier, inc=1, device_id=(tgt,),
                            device_id_type=pl.DeviceIdType.MESH)
        n_peers += 1
    for dr in range(1, _NZ):
        tgt = lax.rem(z + dr, _NZ) * _PLANE + q
        pl.semaphore_signal(barrier, inc=1, device_id=(tgt,),
                            device_id_type=pl.DeviceIdType.MESH)
        n_peers += 1
    pl.semaphore_wait(barrier, n_peers)
    return n_peers


def _hier_body(p_ref, out_ref, rbuf1, rbuf2, send_sems,
               r_sems1, r_sems2, r_sems3, r_sems4):
    me = lax.axis_index("i")
    z = lax.div(me, _PLANE)
    q = lax.rem(me, _PLANE)

    n_peers = _entry_barrier(z, q)

    out_ref[...] = p_ref[...]
    _hier_collective(me, z, q, n_peers, out_ref, rbuf1, rbuf2, send_sems,
                     r_sems1, r_sems2, r_sems3, r_sems4)


def _hier_collective(me, z, q, n_peers, out_ref, rbuf1, rbuf2, send_sems,
                     r_sems1, r_sems2, r_sems3, r_sems4):
    my_blk = pl.multiple_of(q * _BLK1, 8)
    my_chunk = pl.multiple_of(q * _BLK1 + z * _BLK2, 8)

    sends = []
    for dq in range(1, _PLANE):
        qq = lax.rem(q + dq, _PLANE)
        tgt = z * _PLANE + qq
        r = pltpu.make_async_remote_copy(
            src_ref=out_ref.at[pl.ds(pl.multiple_of(qq * _BLK1, 8), _BLK1), :],
            dst_ref=rbuf1.at[pl.ds(my_blk, _BLK1), :],
            send_sem=send_sems.at[dq],
            recv_sem=r_sems1.at[_PLANE - dq],
            device_id=(tgt,),
            device_id_type=pl.DeviceIdType.MESH,
        )
        r.start()
        sends.append(r)
    acc = out_ref[pl.ds(my_blk, _BLK1), :]
    for dq in range(1, _PLANE):
        qq = lax.rem(q + dq, _PLANE)
        w = pltpu.make_async_remote_copy(
            src_ref=rbuf1.at[pl.ds(pl.multiple_of(qq * _BLK1, 8), _BLK1), :],
            dst_ref=rbuf1.at[pl.ds(pl.multiple_of(qq * _BLK1, 8), _BLK1), :],
            send_sem=send_sems.at[dq],
            recv_sem=r_sems1.at[dq],
            device_id=(me,),
            device_id_type=pl.DeviceIdType.MESH,
        )
        w.wait_recv()
        acc = acc + rbuf1[pl.ds(pl.multiple_of(qq * _BLK1, 8), _BLK1), :]
    out_ref[pl.ds(my_blk, _BLK1), :] = acc
    for r in sends:
        r.wait_send()

    sends = []
    for dr in range(1, _NZ):
        rr = lax.rem(z + dr, _NZ)
        tgt = rr * _PLANE + q
        src_off = pl.multiple_of(q * _BLK1 + rr * _BLK2, 8)
        r = pltpu.make_async_remote_copy(
            src_ref=out_ref.at[pl.ds(src_off, _BLK2), :],
            dst_ref=rbuf2.at[pl.ds(pl.multiple_of(z * _BLK2, 8), _BLK2), :],
            send_sem=send_sems.at[dr],
            recv_sem=r_sems2.at[_NZ - dr],
            device_id=(tgt,),
            device_id_type=pl.DeviceIdType.MESH,
        )
        r.start()
        sends.append(r)
    acc = out_ref[pl.ds(my_chunk, _BLK2), :]
    for dr in range(1, _NZ):
        rr = lax.rem(z + dr, _NZ)
        w = pltpu.make_async_remote_copy(
            src_ref=rbuf2.at[pl.ds(pl.multiple_of(rr * _BLK2, 8), _BLK2), :],
            dst_ref=rbuf2.at[pl.ds(pl.multiple_of(rr * _BLK2, 8), _BLK2), :],
            send_sem=send_sems.at[dr],
            recv_sem=r_sems2.at[dr],
            device_id=(me,),
            device_id_type=pl.DeviceIdType.MESH,
        )
        w.wait_recv()
        acc = acc + rbuf2[pl.ds(pl.multiple_of(rr * _BLK2, 8), _BLK2), :]
    out_ref[pl.ds(my_chunk, _BLK2), :] = acc
    for r in sends:
        r.wait_send()

    sends = []
    for dr in range(1, _NZ):
        rr = lax.rem(z + dr, _NZ)
        tgt = rr * _PLANE + q
        r = pltpu.make_async_remote_copy(
            src_ref=out_ref.at[pl.ds(my_chunk, _BLK2), :],
            dst_ref=out_ref.at[pl.ds(my_chunk, _BLK2), :],
            send_sem=send_sems.at[dr],
            recv_sem=r_sems3.at[_NZ - dr],
            device_id=(tgt,),
            device_id_type=pl.DeviceIdType.MESH,
        )
        r.start()
        sends.append(r)
    for dr in range(1, _NZ):
        rr = lax.rem(z + dr, _NZ)
        their = pl.multiple_of(q * _BLK1 + rr * _BLK2, 8)
        w = pltpu.make_async_remote_copy(
            src_ref=out_ref.at[pl.ds(their, _BLK2), :],
            dst_ref=out_ref.at[pl.ds(their, _BLK2), :],
            send_sem=send_sems.at[dr],
            recv_sem=r_sems3.at[dr],
            device_id=(me,),
            device_id_type=pl.DeviceIdType.MESH,
        )
        w.wait_recv()
    for r in sends:
        r.wait_send()

    sends = []
    for dq in range(1, _PLANE):
        qq = lax.rem(q + dq, _PLANE)
        tgt = z * _PLANE + qq
        r = pltpu.make_async_remote_copy(
            src_ref=out_ref.at[pl.ds(my_blk, _BLK1), :],
            dst_ref=out_ref.at[pl.ds(my_blk, _BLK1), :],
            send_sem=send_sems.at[dq],
            recv_sem=r_sems4.at[_PLANE - dq],
            device_id=(tgt,),
            device_id_type=pl.DeviceIdType.MESH,
        )
        r.start()
        sends.append(r)
    for dq in range(1, _PLANE):
        qq = lax.rem(q + dq, _PLANE)
        their = pl.multiple_of(qq * _BLK1, 8)
        w = pltpu.make_async_remote_copy(
            src_ref=out_ref.at[pl.ds(their, _BLK1), :],
            dst_ref=out_ref.at[pl.ds(their, _BLK1), :],
            send_sem=send_sems.at[dq],
            recv_sem=r_sems4.at[dq],
            device_id=(me,),
            device_id_type=pl.DeviceIdType.MESH,
        )
        w.wait_recv()
    for r in sends:
        r.wait_send()

    import functools

    @functools.partial(pl.run_scoped, exit_sem=pltpu.SemaphoreType.REGULAR)
    def _(exit_sem):
        for dq in range(1, _PLANE):
            tgt = z * _PLANE + lax.rem(q + dq, _PLANE)
            pl.semaphore_signal(exit_sem, inc=1, device_id=(tgt,),
                                device_id_type=pl.DeviceIdType.MESH)
        for dr in range(1, _NZ):
            tgt = lax.rem(z + dr, _NZ) * _PLANE + q
            pl.semaphore_signal(exit_sem, inc=1, device_id=(tgt,),
                                device_id_type=pl.DeviceIdType.MESH)
        pl.semaphore_wait(exit_sem, n_peers)


def _flat_body(p_ref, out_ref, rbuf, send_sems, rs_sems, ag_sems):
    me = lax.axis_index("i")

    import os as _os2
    if _os2.environ.get("DBG_PROBE", "") == "minbar":
        with jax.named_scope("entry_barrier"):
            barrier = pltpu.get_barrier_semaphore()
            pl.semaphore_signal(barrier, inc=1,
                                device_id=(lax.rem(me + 1, N_DEV),),
                                device_id_type=pl.DeviceIdType.MESH)
            pl.semaphore_wait(barrier, 1)
    else:
        with jax.named_scope("entry_barrier"):
            barrier = pltpu.get_barrier_semaphore()
            for d in range(1, N_DEV):
                tgt = lax.rem(me + d, N_DEV)
                pl.semaphore_signal(barrier, inc=1, device_id=(tgt,),
                                    device_id_type=pl.DeviceIdType.MESH)
            pl.semaphore_wait(barrier, N_DEV - 1)

    out_ref[...] = p_ref[...]
    my_off = pl.multiple_of(me * CHUNK, 8)

    import os as _os
    _PROBE = _os.environ.get("DBG_PROBE", "")
    if _PROBE in ("barrier", "minbar"):
        return

    sends = []
    with jax.named_scope("rs_issue"):
        for d in range(1, N_DEV):
            tgt = lax.rem(me + d, N_DEV)
            r = pltpu.make_async_remote_copy(
                src_ref=out_ref.at[pl.ds(pl.multiple_of(tgt * CHUNK, 8), CHUNK), :],
                dst_ref=rbuf.at[pl.ds(my_off, CHUNK), :],
                send_sem=send_sems.at[d],
                recv_sem=rs_sems.at[me],
                device_id=(tgt,),
                device_id_type=pl.DeviceIdType.MESH,
            )
            r.start()
            sends.append(r)
        rbuf[pl.ds(my_off, CHUNK), :] = out_ref[pl.ds(my_off, CHUNK), :]
    with jax.named_scope("rs_wait"):
        for d in range(1, N_DEV):
            src = lax.rem(me + d, N_DEV)
            w = pltpu.make_async_remote_copy(
                src_ref=rbuf.at[pl.ds(pl.multiple_of(src * CHUNK, 8), CHUNK), :],
                dst_ref=rbuf.at[pl.ds(pl.multiple_of(src * CHUNK, 8), CHUNK), :],
                send_sem=send_sems.at[d],
                recv_sem=rs_sems.at[src],
                device_id=(me,),
                device_id_type=pl.DeviceIdType.MESH,
            )
            w.wait_recv()
    with jax.named_scope("rs_tree_sum"):
        for half in (128, 64, 32, 16, 8):
            a = rbuf[0:half, :]
            b = rbuf[half:2 * half, :]
            rbuf[0:half, :] = a + b
        out_ref[pl.ds(my_off, CHUNK), :] = rbuf[0:CHUNK, :]
        for r in sends:
            r.wait_send()

    if _PROBE == "rs":
        return

    sends = []
    with jax.named_scope("ag_issue"):
        for d in range(1, N_DEV):
            tgt = lax.rem(me + d, N_DEV)
            r = pltpu.make_async_remote_copy(
                src_ref=out_ref.at[pl.ds(my_off, CHUNK), :],
                dst_ref=out_ref.at[pl.ds(my_off, CHUNK), :],
                send_sem=send_sems.at[d],
                recv_sem=ag_sems.at[me],
                device_id=(tgt,),
                device_id_type=pl.DeviceIdType.MESH,
            )
            r.start()
            sends.append(r)
    with jax.named_scope("ag_wait"):
        for d in range(1, N_DEV):
            src = lax.rem(me + d, N_DEV)
            their = pl.multiple_of(src * CHUNK, 8)
            w = pltpu.make_async_remote_copy(
                src_ref=out_ref.at[pl.ds(their, CHUNK), :],
                dst_ref=out_ref.at[pl.ds(their, CHUNK), :],
                send_sem=send_sems.at[d],
                recv_sem=ag_sems.at[src],
                device_id=(me,),
                device_id_type=pl.DeviceIdType.MESH,
            )
            w.wait_recv()
        for r in sends:
            r.wait_send()


def _flat_allreduce(partial):
    return pl.pallas_call(
        _flat_body,
        out_shape=jax.ShapeDtypeStruct((ROWS, D), jnp.float32),
        in_specs=[pl.BlockSpec(memory_space=pltpu.VMEM)],
        out_specs=pl.BlockSpec(memory_space=pltpu.VMEM),
        scratch_shapes=[
            pltpu.VMEM((ROWS, D), jnp.float32),
            pltpu.SemaphoreType.DMA((N_DEV,)),
            pltpu.SemaphoreType.DMA((N_DEV,)),
            pltpu.SemaphoreType.DMA((N_DEV,)),
        ],
        compiler_params=pltpu.CompilerParams(collective_id=0),
    )(partial)


_C_R, _C_L = 16, 256
_N_RB = ROWS // _C_R
_BF = jnp.bfloat16


def _chunk_offs(c):
    return (pl.multiple_of(lax.rem(c, _N_RB) * _C_R, 16),
            pl.multiple_of(lax.div(c, _N_RB) * _C_L, 128))


def _flat16_body(attn_ref, wo_ref, out_ref, pbuf, rbuf, agbuf,
                 send_sems, rs_sems, ag_sems):
    me = lax.axis_index("i")

    barrier = pltpu.get_barrier_semaphore()
    for d in range(1, N_DEV):
        tgt = lax.rem(me + d, N_DEV)
        pl.semaphore_signal(barrier, inc=1, device_id=(tgt,),
                            device_id_type=pl.DeviceIdType.MESH)
    pl.semaphore_wait(barrier, N_DEV - 1)

    pbuf[...] = jnp.dot(attn_ref[...], wo_ref[...],
                        preferred_element_type=jnp.float32).astype(_BF)

    sends = []
    for d in range(1, N_DEV):
        tgt = lax.rem(me + d, N_DEV)
        ro, lo = _chunk_offs(tgt)
        r = pltpu.make_async_remote_copy(
            src_ref=pbuf.at[pl.ds(ro, _C_R), pl.ds(lo, _C_L)],
            dst_ref=rbuf.at[me],
            send_sem=send_sems.at[d],
            recv_sem=rs_sems.at[me],
            device_id=(tgt,),
            device_id_type=pl.DeviceIdType.MESH,
        )
        r.start()
        sends.append(r)
    my_ro, my_lo = _chunk_offs(me)
    rbuf[me] = pbuf[pl.ds(my_ro, _C_R), pl.ds(my_lo, _C_L)]
    for d in range(1, N_DEV):
        src = lax.rem(me + d, N_DEV)
        w = pltpu.make_async_remote_copy(
            src_ref=rbuf.at[src],
            dst_ref=rbuf.at[src],
            send_sem=send_sems.at[d],
            recv_sem=rs_sems.at[src],
            device_id=(me,),
            device_id_type=pl.DeviceIdType.MESH,
        )
        w.wait_recv()
    for half in (16, 8, 4, 2, 1):
        rbuf[0:half] = rbuf[0:half] + rbuf[half:2 * half]
    for r in sends:
        r.wait_send()

    sends = []
    for d in range(1, N_DEV):
        tgt = lax.rem(me + d, N_DEV)
        r = pltpu.make_async_remote_copy(
            src_ref=rbuf.at[0],
            dst_ref=agbuf.at[me],
            send_sem=send_sems.at[d],
            recv_sem=ag_sems.at[me],
            device_id=(tgt,),
            device_id_type=pl.DeviceIdType.MESH,
        )
        r.start()
        sends.append(r)
    agbuf[me] = rbuf[0]
    for d in range(1, N_DEV):
        src = lax.rem(me + d, N_DEV)
        w = pltpu.make_async_remote_copy(
            src_ref=agbuf.at[src],
            dst_ref=agbuf.at[src],
            send_sem=send_sems.at[d],
            recv_sem=ag_sems.at[src],
            device_id=(me,),
            device_id_type=pl.DeviceIdType.MESH,
        )
        w.wait_recv()
    for c in range(N_DEV):
        ro = (c % _N_RB) * _C_R
        lo = (c // _N_RB) * _C_L
        out_ref[ro:ro + _C_R, lo:lo + _C_L] = agbuf[c].astype(jnp.float32)
    for r in sends:
        r.wait_send()


def _flat16_allreduce(attn, Wo):
    return pl.pallas_call(
        _flat16_body,
        out_shape=jax.ShapeDtypeStruct((ROWS, D), jnp.float32),
        in_specs=[pl.BlockSpec(memory_space=pltpu.VMEM)] * 2,
        out_specs=pl.BlockSpec(memory_space=pltpu.VMEM),
        scratch_shapes=[
            pltpu.VMEM((ROWS, D), _BF),
            pltpu.VMEM((N_DEV, _C_R, _C_L), _BF),
            pltpu.VMEM((N_DEV, _C_R, _C_L), _BF),
            pltpu.SemaphoreType.DMA((N_DEV,)),
            pltpu.SemaphoreType.DMA((N_DEV,)),
            pltpu.SemaphoreType.DMA((N_DEV,)),
        ],
        compiler_params=pltpu.CompilerParams(collective_id=0),
    )(attn, Wo)


def _fused_body(x_ref, wq_ref, wo_ref, wk_ref, wv_ref, out_ref,
                attn_ref, rbuf1, rbuf2, send_sems,
                r_sems1, r_sems2, r_sems3, r_sems4):
    me = lax.axis_index("i")
    z = lax.div(me, _PLANE)
    q = lax.rem(me, _PLANE)

    n_peers = _entry_barrier(z, q)

    x2 = x_ref[...]
    Q = jnp.dot(x2, wq_ref[...], preferred_element_type=jnp.float32)
    kv_col = pl.multiple_of(me * (HQ_SHARD // GQA) * DH, 128)
    K2 = jnp.dot(x2, wk_ref[:, pl.ds(kv_col, (HQ_SHARD // GQA) * DH)],
                 preferred_element_type=jnp.float32)
    V2 = jnp.dot(x2, wv_ref[:, pl.ds(kv_col, (HQ_SHARD // GQA) * DH)],
                 preferred_element_type=jnp.float32)

    for b in range(B):
        rows = slice(b * SQ, (b + 1) * SQ)
        for h in range(HQ_SHARD):
            qh = Q[rows, h * DH:(h + 1) * DH]
            kh = K2[rows, (h // GQA) * DH:(h // GQA + 1) * DH]
            vh = V2[rows, (h // GQA) * DH:(h // GQA + 1) * DH]
            s = lax.dot_general(qh, kh, (((1,), (1,)), ((), ())),
                                preferred_element_type=jnp.float32) * 0.125
            m = jnp.max(s, axis=1, keepdims=True)
            p = jnp.exp(s - m)
            p = p / jnp.sum(p, axis=1, keepdims=True)
            attn_ref[rows, h * DH:(h + 1) * DH] = jnp.dot(
                p, vh, preferred_element_type=jnp.float32)

    out_ref[...] = jnp.dot(attn_ref[...], wo_ref[...],
                           preferred_element_type=jnp.float32)

    _hier_collective(me, z, q, n_peers, out_ref, rbuf1, rbuf2, send_sems,
                     r_sems1, r_sems2, r_sems3, r_sems4)


def _fused_attn_allreduce(x2, Wq, Wo, Wk, Wv):
    return pl.pallas_call(
        _fused_body,
        out_shape=jax.ShapeDtypeStruct((ROWS, D), jnp.float32),
        in_specs=[pl.BlockSpec(memory_space=pltpu.VMEM)] * 5,
        out_specs=pl.BlockSpec(memory_space=pltpu.VMEM),
        scratch_shapes=[
            pltpu.VMEM((ROWS, HQ_SHARD * DH), jnp.float32),
            pltpu.VMEM((ROWS, D), jnp.float32),
            pltpu.VMEM((_BLK1, D), jnp.float32),
            pltpu.SemaphoreType.DMA((_PLANE,)),
            pltpu.SemaphoreType.DMA((_PLANE,)),
            pltpu.SemaphoreType.DMA((_NZ,)),
            pltpu.SemaphoreType.DMA((_NZ,)),
            pltpu.SemaphoreType.DMA((_PLANE,)),
        ],
        compiler_params=pltpu.CompilerParams(collective_id=0),
    )(x2, Wq, Wo, Wk, Wv)


def _hier_allreduce(partial):
    return pl.pallas_call(
        _hier_body,
        out_shape=jax.ShapeDtypeStruct((ROWS, D), jnp.float32),
        in_specs=[pl.BlockSpec(memory_space=pltpu.VMEM)],
        out_specs=pl.BlockSpec(memory_space=pltpu.VMEM),
        scratch_shapes=[
            pltpu.VMEM((ROWS, D), jnp.float32),
            pltpu.VMEM((_BLK1, D), jnp.float32),
            pltpu.SemaphoreType.DMA((_PLANE,)),
            pltpu.SemaphoreType.DMA((_PLANE,)),
            pltpu.SemaphoreType.DMA((_NZ,)),
            pltpu.SemaphoreType.DMA((_NZ,)),
            pltpu.SemaphoreType.DMA((_PLANE,)),
        ],
        compiler_params=pltpu.CompilerParams(collective_id=0),
    )(partial)


_RS_MASKS = (16, 8, 4, 2, 1)
_AG_MASKS = (1, 2, 4, 8, 16)
_RS_SIZES = (128, 64, 32, 16, 8)
_AG_SIZES = (8, 16, 32, 64, 128)
_RS_OFF = (0, 128, 192, 224, 240)
_AG_OFF = (248, 256, 272, 304, 368)
_RECV_ROWS = 496


def _butterfly_body(p_ref, out_ref, recvbuf, send_sems, recv_sems):
    me = lax.axis_index("i")
    partners = [jnp.bitwise_xor(me, m) for m in (1, 2, 4, 8, 16)]

    barrier = pltpu.get_barrier_semaphore()
    for p in partners:
        pl.semaphore_signal(barrier, inc=1, device_id=(p,),
                            device_id_type=pl.DeviceIdType.MESH)
    pl.semaphore_wait(barrier, len(partners))

    out_ref[...] = p_ref[...]

    off = jnp.int32(0)
    for k, m in enumerate(_RS_MASKS):
        partner = jnp.bitwise_xor(me, m)
        half = _RS_SIZES[k]
        keep_low = jnp.bitwise_and(me, m) == 0
        send_off = pl.multiple_of(off + jnp.where(keep_low, half, 0), 8)
        keep_off = pl.multiple_of(off + jnp.where(keep_low, 0, half), 8)
        rdma = pltpu.make_async_remote_copy(
            src_ref=out_ref.at[pl.ds(send_off, half), :],
            dst_ref=recvbuf.at[pl.ds(_RS_OFF[k], half), :],
            send_sem=send_sems.at[k],
            recv_sem=recv_sems.at[k],
            device_id=(partner,),
            device_id_type=pl.DeviceIdType.MESH,
        )
        rdma.start()
        rdma.wait()
        cur = out_ref[pl.ds(keep_off, half), :]
        out_ref[pl.ds(keep_off, half), :] = (
            cur + recvbuf[pl.ds(_RS_OFF[k], half), :]
        )
        off = keep_off

    for j, m in enumerate(_AG_MASKS):
        k = 5 + j
        partner = jnp.bitwise_xor(me, m)
        sz = _AG_SIZES[j]
        their_off = pl.multiple_of(jnp.bitwise_xor(off, 8 * m), 8)
        rdma = pltpu.make_async_remote_copy(
            src_ref=out_ref.at[pl.ds(pl.multiple_of(off, 8), sz), :],
            dst_ref=recvbuf.at[pl.ds(_AG_OFF[j], sz), :],
            send_sem=send_sems.at[k],
            recv_sem=recv_sems.at[k],
            device_id=(partner,),
            device_id_type=pl.DeviceIdType.MESH,
        )
        rdma.start()
        rdma.wait()
        out_ref[pl.ds(their_off, sz), :] = recvbuf[pl.ds(_AG_OFF[j], sz), :]
        off = jnp.bitwise_and(off, jnp.int32(~(8 * m)))

    import functools

    @functools.partial(pl.run_scoped, exit_sem=pltpu.SemaphoreType.REGULAR)
    def _(exit_sem):
        for p in partners:
            pl.semaphore_signal(exit_sem, inc=1, device_id=(p,),
                                device_id_type=pl.DeviceIdType.MESH)
        pl.semaphore_wait(exit_sem, len(partners))


def _butterfly_allreduce(partial):
    return pl.pallas_call(
        _butterfly_body,
        out_shape=jax.ShapeDtypeStruct((ROWS, D), jnp.float32),
        in_specs=[pl.BlockSpec(memory_space=pltpu.VMEM)],
        out_specs=pl.BlockSpec(memory_space=pltpu.VMEM),
        scratch_shapes=[
            pltpu.VMEM((_RECV_ROWS, D), jnp.float32),
            pltpu.SemaphoreType.DMA((10,)),
            pltpu.SemaphoreType.DMA((10,)),
        ],
        compiler_params=pltpu.CompilerParams(collective_id=0),
    )(partial)


def _ring_allreduce(partial):
    return pl.pallas_call(
        _allreduce_body,
        out_shape=jax.ShapeDtypeStruct((ROWS, D), jnp.float32),
        in_specs=[pl.BlockSpec(memory_space=pltpu.VMEM)],
        out_specs=pl.BlockSpec(memory_space=pltpu.VMEM),
        scratch_shapes=[
            pltpu.VMEM((2, CHUNK, D), jnp.float32),
            pltpu.VMEM((2, CHUNK, D), jnp.float32),
            pltpu.SemaphoreType.DMA((2,)),
            pltpu.SemaphoreType.DMA((2,)),
            pltpu.SemaphoreType.REGULAR,
        ],
        compiler_params=pltpu.CompilerParams(collective_id=0),
    )(partial)


def kernel(x, Wq, Wo, Wk, Wv):
    i = lax.axis_index("i")
    x2 = x.reshape(ROWS, D)

    Q = (x2 @ Wq).reshape(B, SQ, HQ_SHARD, DH)

    kv_cols = (HQ_SHARD // GQA) * DH
    Wk_s = lax.dynamic_slice_in_dim(Wk, i * kv_cols, kv_cols, axis=1)
    Wv_s = lax.dynamic_slice_in_dim(Wv, i * kv_cols, kv_cols, axis=1)
    K = (x2 @ Wk_s).reshape(B, SQ, HQ_SHARD // GQA, DH)
    V = (x2 @ Wv_s).reshape(B, SQ, HQ_SHARD // GQA, DH)
    K = jnp.repeat(K, GQA, axis=2)
    V = jnp.repeat(V, GQA, axis=2)

    s = jnp.einsum("bihd,bjhd->bhij", Q, K) * 0.125
    p = jax.nn.softmax(s, axis=-1)
    attn = jnp.einsum("bhij,bjhd->bihd", p, V).reshape(ROWS, HQ_SHARD * DH)

    out = _flat16_allreduce(attn, Wo)
    return out.reshape(B, SQ, D)


# device time: 29830 ns/iter; 1.0008x vs baseline; 1.0008x over previous
import jax
import jax.numpy as jnp
from jax import lax
from jax.experimental import pallas as pl
from jax.experimental.pallas import tpu as pltpu

N_DEV = 32
B, SQ, D = 2, 128, 512
HQ_SHARD = 8
GQA = 4
DH = 64
ROWS = B * SQ
CHUNK = ROWS // N_DEV


def _allreduce_body(p_ref, out_ref, sendbuf, recvbuf, send_sems, recv_sems,
                    credit_sem):
    me = lax.axis_index("i")
    left = lax.rem(me + N_DEV - 1, N_DEV)
    right = lax.rem(me + 1, N_DEV)

    barrier = pltpu.get_barrier_semaphore()
    for nbr in (left, right):
        pl.semaphore_signal(barrier, inc=1, device_id=(nbr,),
                            device_id_type=pl.DeviceIdType.MESH)
    pl.semaphore_wait(barrier, 2)

    out_ref[...] = p_ref[...]

    total_steps = 2 * (N_DEV - 1)
    for h in range(total_steps):
        slot = h % 2
        rs_phase = h < N_DEV - 1
        if rs_phase:
            c_send = lax.rem(me - h + 2 * N_DEV, N_DEV)
            c_recv = lax.rem(me - h - 1 + 2 * N_DEV, N_DEV)
        else:
            g = h - (N_DEV - 1)
            c_send = lax.rem(me + 1 - g + 2 * N_DEV, N_DEV)
            c_recv = lax.rem(me - g + 2 * N_DEV, N_DEV)

        sendbuf[slot] = out_ref[pl.ds(c_send * CHUNK, CHUNK), :]

        if h >= 2:
            pl.semaphore_wait(credit_sem, 1)

        rdma = pltpu.make_async_remote_copy(
            src_ref=sendbuf.at[slot],
            dst_ref=recvbuf.at[slot],
            send_sem=send_sems.at[slot],
            recv_sem=recv_sems.at[slot],
            device_id=(right,),
            device_id_type=pl.DeviceIdType.MESH,
        )
        rdma.start()
        rdma.wait()

        if rs_phase:
            cur = out_ref[pl.ds(c_recv * CHUNK, CHUNK), :]
            out_ref[pl.ds(c_recv * CHUNK, CHUNK), :] = cur + recvbuf[slot]
        else:
            out_ref[pl.ds(c_recv * CHUNK, CHUNK), :] = recvbuf[slot]

        if h < total_steps - 2:
            pl.semaphore_signal(credit_sem, inc=1, device_id=(left,),
                                device_id_type=pl.DeviceIdType.MESH)


_PLANE = 8
_NZ = 4
_BLK1 = ROWS // _PLANE
_BLK2 = _BLK1 // _NZ


def _entry_barrier(z, q):
    barrier = pltpu.get_barrier_semaphore()
    n_peers = 0
    for dq in range(1, _PLANE):
        tgt = z * _PLANE + lax.rem(q + dq, _PLANE)
        pl.semaphore_signal(barrier, inc=1, device_id=(tgt,),
                            device_id_type=pl.DeviceIdType.MESH)
        n_peers += 1
    for dr in range(1, _NZ):
        tgt = lax.rem(z + dr, _NZ) * _PLANE + q
        pl.semaphore_signal(barrier, inc=1, device_id=(tgt,),
                            device_id_type=pl.DeviceIdType.MESH)
        n_peers += 1
    pl.semaphore_wait(barrier, n_peers)
    return n_peers


def _hier_body(p_ref, out_ref, rbuf1, rbuf2, send_sems,
               r_sems1, r_sems2, r_sems3, r_sems4):
    me = lax.axis_index("i")
    z = lax.div(me, _PLANE)
    q = lax.rem(me, _PLANE)

    n_peers = _entry_barrier(z, q)

    out_ref[...] = p_ref[...]
    _hier_collective(me, z, q, n_peers, out_ref, rbuf1, rbuf2, send_sems,
                     r_sems1, r_sems2, r_sems3, r_sems4)


def _hier_collective(me, z, q, n_peers, out_ref, rbuf1, rbuf2, send_sems,
                     r_sems1, r_sems2, r_sems3, r_sems4):
    my_blk = pl.multiple_of(q * _BLK1, 8)
    my_chunk = pl.multiple_of(q * _BLK1 + z * _BLK2, 8)

    sends = []
    for dq in range(1, _PLANE):
        qq = lax.rem(q + dq, _PLANE)
        tgt = z * _PLANE + qq
        r = pltpu.make_async_remote_copy(
            src_ref=out_ref.at[pl.ds(pl.multiple_of(qq * _BLK1, 8), _BLK1), :],
            dst_ref=rbuf1.at[pl.ds(my_blk, _BLK1), :],
            send_sem=send_sems.at[dq],
            recv_sem=r_sems1.at[_PLANE - dq],
            device_id=(tgt,),
            device_id_type=pl.DeviceIdType.MESH,
        )
        r.start()
        sends.append(r)
    acc = out_ref[pl.ds(my_blk, _BLK1), :]
    for dq in range(1, _PLANE):
        qq = lax.rem(q + dq, _PLANE)
        w = pltpu.make_async_remote_copy(
            src_ref=rbuf1.at[pl.ds(pl.multiple_of(qq * _BLK1, 8), _BLK1), :],
            dst_ref=rbuf1.at[pl.ds(pl.multiple_of(qq * _BLK1, 8), _BLK1), :],
            send_sem=send_sems.at[dq],
            recv_sem=r_sems1.at[dq],
            device_id=(me,),
            device_id_type=pl.DeviceIdType.MESH,
        )
        w.wait_recv()
        acc = acc + rbuf1[pl.ds(pl.multiple_of(qq * _BLK1, 8), _BLK1), :]
    out_ref[pl.ds(my_blk, _BLK1), :] = acc
    for r in sends:
        r.wait_send()

    sends = []
    for dr in range(1, _NZ):
        rr = lax.rem(z + dr, _NZ)
        tgt = rr * _PLANE + q
        src_off = pl.multiple_of(q * _BLK1 + rr * _BLK2, 8)
        r = pltpu.make_async_remote_copy(
            src_ref=out_ref.at[pl.ds(src_off, _BLK2), :],
            dst_ref=rbuf2.at[pl.ds(pl.multiple_of(z * _BLK2, 8), _BLK2), :],
            send_sem=send_sems.at[dr],
            recv_sem=r_sems2.at[_NZ - dr],
            device_id=(tgt,),
            device_id_type=pl.DeviceIdType.MESH,
        )
        r.start()
        sends.append(r)
    acc = out_ref[pl.ds(my_chunk, _BLK2), :]
    for dr in range(1, _NZ):
        rr = lax.rem(z + dr, _NZ)
        w = pltpu.make_async_remote_copy(
            src_ref=rbuf2.at[pl.ds(pl.multiple_of(rr * _BLK2, 8), _BLK2), :],
            dst_ref=rbuf2.at[pl.ds(pl.multiple_of(rr * _BLK2, 8), _BLK2), :],
            send_sem=send_sems.at[dr],
            recv_sem=r_sems2.at[dr],
            device_id=(me,),
            device_id_type=pl.DeviceIdType.MESH,
        )
        w.wait_recv()
        acc = acc + rbuf2[pl.ds(pl.multiple_of(rr * _BLK2, 8), _BLK2), :]
    out_ref[pl.ds(my_chunk, _BLK2), :] = acc
    for r in sends:
        r.wait_send()

    sends = []
    for dr in range(1, _NZ):
        rr = lax.rem(z + dr, _NZ)
        tgt = rr * _PLANE + q
        r = pltpu.make_async_remote_copy(
            src_ref=out_ref.at[pl.ds(my_chunk, _BLK2), :],
            dst_ref=out_ref.at[pl.ds(my_chunk, _BLK2), :],
            send_sem=send_sems.at[dr],
            recv_sem=r_sems3.at[_NZ - dr],
            device_id=(tgt,),
            device_id_type=pl.DeviceIdType.MESH,
        )
        r.start()
        sends.append(r)
    for dr in range(1, _NZ):
        rr = lax.rem(z + dr, _NZ)
        their = pl.multiple_of(q * _BLK1 + rr * _BLK2, 8)
        w = pltpu.make_async_remote_copy(
            src_ref=out_ref.at[pl.ds(their, _BLK2), :],
            dst_ref=out_ref.at[pl.ds(their, _BLK2), :],
            send_sem=send_sems.at[dr],
            recv_sem=r_sems3.at[dr],
            device_id=(me,),
            device_id_type=pl.DeviceIdType.MESH,
        )
        w.wait_recv()
    for r in sends:
        r.wait_send()

    sends = []
    for dq in range(1, _PLANE):
        qq = lax.rem(q + dq, _PLANE)
        tgt = z * _PLANE + qq
        r = pltpu.make_async_remote_copy(
            src_ref=out_ref.at[pl.ds(my_blk, _BLK1), :],
            dst_ref=out_ref.at[pl.ds(my_blk, _BLK1), :],
            send_sem=send_sems.at[dq],
            recv_sem=r_sems4.at[_PLANE - dq],
            device_id=(tgt,),
            device_id_type=pl.DeviceIdType.MESH,
        )
        r.start()
        sends.append(r)
    for dq in range(1, _PLANE):
        qq = lax.rem(q + dq, _PLANE)
        their = pl.multiple_of(qq * _BLK1, 8)
        w = pltpu.make_async_remote_copy(
            src_ref=out_ref.at[pl.ds(their, _BLK1), :],
            dst_ref=out_ref.at[pl.ds(their, _BLK1), :],
            send_sem=send_sems.at[dq],
            recv_sem=r_sems4.at[dq],
            device_id=(me,),
            device_id_type=pl.DeviceIdType.MESH,
        )
        w.wait_recv()
    for r in sends:
        r.wait_send()

    import functools

    @functools.partial(pl.run_scoped, exit_sem=pltpu.SemaphoreType.REGULAR)
    def _(exit_sem):
        for dq in range(1, _PLANE):
            tgt = z * _PLANE + lax.rem(q + dq, _PLANE)
            pl.semaphore_signal(exit_sem, inc=1, device_id=(tgt,),
                                device_id_type=pl.DeviceIdType.MESH)
        for dr in range(1, _NZ):
            tgt = lax.rem(z + dr, _NZ) * _PLANE + q
            pl.semaphore_signal(exit_sem, inc=1, device_id=(tgt,),
                                device_id_type=pl.DeviceIdType.MESH)
        pl.semaphore_wait(exit_sem, n_peers)


def _flat_body(p_ref, out_ref, rbuf, send_sems, rs_sems, ag_sems):
    me = lax.axis_index("i")

    import os as _os2
    if _os2.environ.get("DBG_PROBE", "") == "minbar":
        with jax.named_scope("entry_barrier"):
            barrier = pltpu.get_barrier_semaphore()
            pl.semaphore_signal(barrier, inc=1,
                                device_id=(lax.rem(me + 1, N_DEV),),
                                device_id_type=pl.DeviceIdType.MESH)
            pl.semaphore_wait(barrier, 1)
    else:
        with jax.named_scope("entry_barrier"):
            barrier = pltpu.get_barrier_semaphore()
            for d in range(1, N_DEV):
                tgt = lax.rem(me + d, N_DEV)
                pl.semaphore_signal(barrier, inc=1, device_id=(tgt,),
                                    device_id_type=pl.DeviceIdType.MESH)
            pl.semaphore_wait(barrier, N_DEV - 1)

    out_ref[...] = p_ref[...]
    my_off = pl.multiple_of(me * CHUNK, 8)

    import os as _os
    _PROBE = _os.environ.get("DBG_PROBE", "")
    if _PROBE in ("barrier", "minbar"):
        return

    sends = []
    with jax.named_scope("rs_issue"):
        for d in range(1, N_DEV):
            tgt = lax.rem(me + d, N_DEV)
            r = pltpu.make_async_remote_copy(
                src_ref=out_ref.at[pl.ds(pl.multiple_of(tgt * CHUNK, 8), CHUNK), :],
                dst_ref=rbuf.at[pl.ds(my_off, CHUNK), :],
                send_sem=send_sems.at[d],
                recv_sem=rs_sems.at[me],
                device_id=(tgt,),
                device_id_type=pl.DeviceIdType.MESH,
            )
            r.start()
            sends.append(r)
        rbuf[pl.ds(my_off, CHUNK), :] = out_ref[pl.ds(my_off, CHUNK), :]
    with jax.named_scope("rs_wait"):
        for d in range(1, N_DEV):
            src = lax.rem(me + d, N_DEV)
            w = pltpu.make_async_remote_copy(
                src_ref=rbuf.at[pl.ds(pl.multiple_of(src * CHUNK, 8), CHUNK), :],
                dst_ref=rbuf.at[pl.ds(pl.multiple_of(src * CHUNK, 8), CHUNK), :],
                send_sem=send_sems.at[d],
                recv_sem=rs_sems.at[src],
                device_id=(me,),
                device_id_type=pl.DeviceIdType.MESH,
            )
            w.wait_recv()
    with jax.named_scope("rs_tree_sum"):
        for half in (128, 64, 32, 16, 8):
            a = rbuf[0:half, :]
            b = rbuf[half:2 * half, :]
            rbuf[0:half, :] = a + b
        out_ref[pl.ds(my_off, CHUNK), :] = rbuf[0:CHUNK, :]
        for r in sends:
            r.wait_send()

    if _PROBE == "rs":
        return

    sends = []
    with jax.named_scope("ag_issue"):
        for d in range(1, N_DEV):
            tgt = lax.rem(me + d, N_DEV)
            r = pltpu.make_async_remote_copy(
                src_ref=out_ref.at[pl.ds(my_off, CHUNK), :],
                dst_ref=out_ref.at[pl.ds(my_off, CHUNK), :],
                send_sem=send_sems.at[d],
                recv_sem=ag_sems.at[me],
                device_id=(tgt,),
                device_id_type=pl.DeviceIdType.MESH,
            )
            r.start()
            sends.append(r)
    with jax.named_scope("ag_wait"):
        for d in range(1, N_DEV):
            src = lax.rem(me + d, N_DEV)
            their = pl.multiple_of(src * CHUNK, 8)
            w = pltpu.make_async_remote_copy(
                src_ref=out_ref.at[pl.ds(their, CHUNK), :],
                dst_ref=out_ref.at[pl.ds(their, CHUNK), :],
                send_sem=send_sems.at[d],
                recv_sem=ag_sems.at[src],
                device_id=(me,),
                device_id_type=pl.DeviceIdType.MESH,
            )
            w.wait_recv()
        for r in sends:
            r.wait_send()


def _flat_allreduce(partial):
    return pl.pallas_call(
        _flat_body,
        out_shape=jax.ShapeDtypeStruct((ROWS, D), jnp.float32),
        in_specs=[pl.BlockSpec(memory_space=pltpu.VMEM)],
        out_specs=pl.BlockSpec(memory_space=pltpu.VMEM),
        scratch_shapes=[
            pltpu.VMEM((ROWS, D), jnp.float32),
            pltpu.SemaphoreType.DMA((N_DEV,)),
            pltpu.SemaphoreType.DMA((N_DEV,)),
            pltpu.SemaphoreType.DMA((N_DEV,)),
        ],
        compiler_params=pltpu.CompilerParams(collective_id=0),
    )(partial)


_C_R, _C_L = 16, 256
_N_RB = ROWS // _C_R
_BF = jnp.bfloat16


def _chunk_offs(c):
    return (pl.multiple_of(lax.rem(c, _N_RB) * _C_R, 16),
            pl.multiple_of(lax.div(c, _N_RB) * _C_L, 128))


def _flat16_body(attn_ref, wo_ref, out_ref, pbuf, rbuf, agbuf,
                 send_sems, rs_sems, ag_sems):
    me = lax.axis_index("i")

    pbuf[...] = jnp.dot(attn_ref[...], wo_ref[...],
                        preferred_element_type=jnp.float32).astype(_BF)

    barrier = pltpu.get_barrier_semaphore()
    for d in range(1, N_DEV):
        tgt = lax.rem(me + d, N_DEV)
        pl.semaphore_signal(barrier, inc=1, device_id=(tgt,),
                            device_id_type=pl.DeviceIdType.MESH)
    pl.semaphore_wait(barrier, N_DEV - 1)

    sends = []
    for d in range(1, N_DEV):
        tgt = lax.rem(me + d, N_DEV)
        ro, lo = _chunk_offs(tgt)
        r = pltpu.make_async_remote_copy(
            src_ref=pbuf.at[pl.ds(ro, _C_R), pl.ds(lo, _C_L)],
            dst_ref=rbuf.at[me],
            send_sem=send_sems.at[d],
            recv_sem=rs_sems.at[me],
            device_id=(tgt,),
            device_id_type=pl.DeviceIdType.MESH,
        )
        r.start()
        sends.append(r)
    my_ro, my_lo = _chunk_offs(me)
    rbuf[me] = pbuf[pl.ds(my_ro, _C_R), pl.ds(my_lo, _C_L)]
    for d in range(1, N_DEV):
        src = lax.rem(me + d, N_DEV)
        w = pltpu.make_async_remote_copy(
            src_ref=rbuf.at[src],
            dst_ref=rbuf.at[src],
            send_sem=send_sems.at[d],
            recv_sem=rs_sems.at[src],
            device_id=(me,),
            device_id_type=pl.DeviceIdType.MESH,
        )
        w.wait_recv()
    for half in (16, 8, 4, 2, 1):
        rbuf[0:half] = rbuf[0:half] + rbuf[half:2 * half]
    for r in sends:
        r.wait_send()

    sends = []
    for d in range(1, N_DEV):
        tgt = lax.rem(me + d, N_DEV)
        r = pltpu.make_async_remote_copy(
            src_ref=rbuf.at[0],
            dst_ref=agbuf.at[me],
            send_sem=send_sems.at[d],
            recv_sem=ag_sems.at[me],
            device_id=(tgt,),
            device_id_type=pl.DeviceIdType.MESH,
        )
        r.start()
        sends.append(r)
    agbuf[me] = rbuf[0]
    for d in range(1, N_DEV):
        src = lax.rem(me + d, N_DEV)
        w = pltpu.make_async_remote_copy(
            src_ref=agbuf.at[src],
            dst_ref=agbuf.at[src],
            send_sem=send_sems.at[d],
            recv_sem=ag_sems.at[src],
            device_id=(me,),
            device_id_type=pl.DeviceIdType.MESH,
        )
        w.wait_recv()
    for c in range(N_DEV):
        ro = (c % _N_RB) * _C_R
        lo = (c // _N_RB) * _C_L
        out_ref[ro:ro + _C_R, lo:lo + _C_L] = agbuf[c].astype(jnp.float32)
    for r in sends:
        r.wait_send()


def _flat16_allreduce(attn, Wo):
    return pl.pallas_call(
        _flat16_body,
        out_shape=jax.ShapeDtypeStruct((ROWS, D), jnp.float32),
        in_specs=[pl.BlockSpec(memory_space=pltpu.VMEM)] * 2,
        out_specs=pl.BlockSpec(memory_space=pltpu.VMEM),
        scratch_shapes=[
            pltpu.VMEM((ROWS, D), _BF),
            pltpu.VMEM((N_DEV, _C_R, _C_L), _BF),
            pltpu.VMEM((N_DEV, _C_R, _C_L), _BF),
            pltpu.SemaphoreType.DMA((N_DEV,)),
            pltpu.SemaphoreType.DMA((N_DEV,)),
            pltpu.SemaphoreType.DMA((N_DEV,)),
        ],
        compiler_params=pltpu.CompilerParams(collective_id=0),
    )(attn, Wo)


def _fused_body(x_ref, wq_ref, wo_ref, wk_ref, wv_ref, out_ref,
                attn_ref, rbuf1, rbuf2, send_sems,
                r_sems1, r_sems2, r_sems3, r_sems4):
    me = lax.axis_index("i")
    z = lax.div(me, _PLANE)
    q = lax.rem(me, _PLANE)

    n_peers = _entry_barrier(z, q)

    x2 = x_ref[...]
    Q = jnp.dot(x2, wq_ref[...], preferred_element_type=jnp.float32)
    kv_col = pl.multiple_of(me * (HQ_SHARD // GQA) * DH, 128)
    K2 = jnp.dot(x2, wk_ref[:, pl.ds(kv_col, (HQ_SHARD // GQA) * DH)],
                 preferred_element_type=jnp.float32)
    V2 = jnp.dot(x2, wv_ref[:, pl.ds(kv_col, (HQ_SHARD // GQA) * DH)],
                 preferred_element_type=jnp.float32)

    for b in range(B):
        rows = slice(b * SQ, (b + 1) * SQ)
        for h in range(HQ_SHARD):
            qh = Q[rows, h * DH:(h + 1) * DH]
            kh = K2[rows, (h // GQA) * DH:(h // GQA + 1) * DH]
            vh = V2[rows, (h // GQA) * DH:(h // GQA + 1) * DH]
            s = lax.dot_general(qh, kh, (((1,), (1,)), ((), ())),
                                preferred_element_type=jnp.float32) * 0.125
            m = jnp.max(s, axis=1, keepdims=True)
            p = jnp.exp(s - m)
            p = p / jnp.sum(p, axis=1, keepdims=True)
            attn_ref[rows, h * DH:(h + 1) * DH] = jnp.dot(
                p, vh, preferred_element_type=jnp.float32)

    out_ref[...] = jnp.dot(attn_ref[...], wo_ref[...],
                           preferred_element_type=jnp.float32)

    _hier_collective(me, z, q, n_peers, out_ref, rbuf1, rbuf2, send_sems,
                     r_sems1, r_sems2, r_sems3, r_sems4)


def _fused_attn_allreduce(x2, Wq, Wo, Wk, Wv):
    return pl.pallas_call(
        _fused_body,
        out_shape=jax.ShapeDtypeStruct((ROWS, D), jnp.float32),
        in_specs=[pl.BlockSpec(memory_space=pltpu.VMEM)] * 5,
        out_specs=pl.BlockSpec(memory_space=pltpu.VMEM),
        scratch_shapes=[
            pltpu.VMEM((ROWS, HQ_SHARD * DH), jnp.float32),
            pltpu.VMEM((ROWS, D), jnp.float32),
            pltpu.VMEM((_BLK1, D), jnp.float32),
            pltpu.SemaphoreType.DMA((_PLANE,)),
            pltpu.SemaphoreType.DMA((_PLANE,)),
            pltpu.SemaphoreType.DMA((_NZ,)),
            pltpu.SemaphoreType.DMA((_NZ,)),
            pltpu.SemaphoreType.DMA((_PLANE,)),
        ],
        compiler_params=pltpu.CompilerParams(collective_id=0),
    )(x2, Wq, Wo, Wk, Wv)


def _hier_allreduce(partial):
    return pl.pallas_call(
        _hier_body,
        out_shape=jax.ShapeDtypeStruct((ROWS, D), jnp.float32),
        in_specs=[pl.BlockSpec(memory_space=pltpu.VMEM)],
        out_specs=pl.BlockSpec(memory_space=pltpu.VMEM),
        scratch_shapes=[
            pltpu.VMEM((ROWS, D), jnp.float32),
            pltpu.VMEM((_BLK1, D), jnp.float32),
            pltpu.SemaphoreType.DMA((_PLANE,)),
            pltpu.SemaphoreType.DMA((_PLANE,)),
            pltpu.SemaphoreType.DMA((_NZ,)),
            pltpu.SemaphoreType.DMA((_NZ,)),
            pltpu.SemaphoreType.DMA((_PLANE,)),
        ],
        compiler_params=pltpu.CompilerParams(collective_id=0),
    )(partial)


_RS_MASKS = (16, 8, 4, 2, 1)
_AG_MASKS = (1, 2, 4, 8, 16)
_RS_SIZES = (128, 64, 32, 16, 8)
_AG_SIZES = (8, 16, 32, 64, 128)
_RS_OFF = (0, 128, 192, 224, 240)
_AG_OFF = (248, 256, 272, 304, 368)
_RECV_ROWS = 496


def _butterfly_body(p_ref, out_ref, recvbuf, send_sems, recv_sems):
    me = lax.axis_index("i")
    partners = [jnp.bitwise_xor(me, m) for m in (1, 2, 4, 8, 16)]

    barrier = pltpu.get_barrier_semaphore()
    for p in partners:
        pl.semaphore_signal(barrier, inc=1, device_id=(p,),
                            device_id_type=pl.DeviceIdType.MESH)
    pl.semaphore_wait(barrier, len(partners))

    out_ref[...] = p_ref[...]

    off = jnp.int32(0)
    for k, m in enumerate(_RS_MASKS):
        partner = jnp.bitwise_xor(me, m)
        half = _RS_SIZES[k]
        keep_low = jnp.bitwise_and(me, m) == 0
        send_off = pl.multiple_of(off + jnp.where(keep_low, half, 0), 8)
        keep_off = pl.multiple_of(off + jnp.where(keep_low, 0, half), 8)
        rdma = pltpu.make_async_remote_copy(
            src_ref=out_ref.at[pl.ds(send_off, half), :],
            dst_ref=recvbuf.at[pl.ds(_RS_OFF[k], half), :],
            send_sem=send_sems.at[k],
            recv_sem=recv_sems.at[k],
            device_id=(partner,),
            device_id_type=pl.DeviceIdType.MESH,
        )
        rdma.start()
        rdma.wait()
        cur = out_ref[pl.ds(keep_off, half), :]
        out_ref[pl.ds(keep_off, half), :] = (
            cur + recvbuf[pl.ds(_RS_OFF[k], half), :]
        )
        off = keep_off

    for j, m in enumerate(_AG_MASKS):
        k = 5 + j
        partner = jnp.bitwise_xor(me, m)
        sz = _AG_SIZES[j]
        their_off = pl.multiple_of(jnp.bitwise_xor(off, 8 * m), 8)
        rdma = pltpu.make_async_remote_copy(
            src_ref=out_ref.at[pl.ds(pl.multiple_of(off, 8), sz), :],
            dst_ref=recvbuf.at[pl.ds(_AG_OFF[j], sz), :],
            send_sem=send_sems.at[k],
            recv_sem=recv_sems.at[k],
            device_id=(partner,),
            device_id_type=pl.DeviceIdType.MESH,
        )
        rdma.start()
        rdma.wait()
        out_ref[pl.ds(their_off, sz), :] = recvbuf[pl.ds(_AG_OFF[j], sz), :]
        off = jnp.bitwise_and(off, jnp.int32(~(8 * m)))

    import functools

    @functools.partial(pl.run_scoped, exit_sem=pltpu.SemaphoreType.REGULAR)
    def _(exit_sem):
        for p in partners:
            pl.semaphore_signal(exit_sem, inc=1, device_id=(p,),
                                device_id_type=pl.DeviceIdType.MESH)
        pl.semaphore_wait(exit_sem, len(partners))


def _butterfly_allreduce(partial):
    return pl.pallas_call(
        _butterfly_body,
        out_shape=jax.ShapeDtypeStruct((ROWS, D), jnp.float32),
        in_specs=[pl.BlockSpec(memory_space=pltpu.VMEM)],
        out_specs=pl.BlockSpec(memory_space=pltpu.VMEM),
        scratch_shapes=[
            pltpu.VMEM((_RECV_ROWS, D), jnp.float32),
            pltpu.SemaphoreType.DMA((10,)),
            pltpu.SemaphoreType.DMA((10,)),
        ],
        compiler_params=pltpu.CompilerParams(collective_id=0),
    )(partial)


def _ring_allreduce(partial):
    return pl.pallas_call(
        _allreduce_body,
        out_shape=jax.ShapeDtypeStruct((ROWS, D), jnp.float32),
        in_specs=[pl.BlockSpec(memory_space=pltpu.VMEM)],
        out_specs=pl.BlockSpec(memory_space=pltpu.VMEM),
        scratch_shapes=[
            pltpu.VMEM((2, CHUNK, D), jnp.float32),
            pltpu.VMEM((2, CHUNK, D), jnp.float32),
            pltpu.SemaphoreType.DMA((2,)),
            pltpu.SemaphoreType.DMA((2,)),
            pltpu.SemaphoreType.REGULAR,
        ],
        compiler_params=pltpu.CompilerParams(collective_id=0),
    )(partial)


def kernel(x, Wq, Wo, Wk, Wv):
    i = lax.axis_index("i")
    x2 = x.reshape(ROWS, D)

    Q = (x2 @ Wq).reshape(B, SQ, HQ_SHARD, DH)

    kv_cols = (HQ_SHARD // GQA) * DH
    Wk_s = lax.dynamic_slice_in_dim(Wk, i * kv_cols, kv_cols, axis=1)
    Wv_s = lax.dynamic_slice_in_dim(Wv, i * kv_cols, kv_cols, axis=1)
    K = (x2 @ Wk_s).reshape(B, SQ, HQ_SHARD // GQA, DH)
    V = (x2 @ Wv_s).reshape(B, SQ, HQ_SHARD // GQA, DH)
    K = jnp.repeat(K, GQA, axis=2)
    V = jnp.repeat(V, GQA, axis=2)

    s = jnp.einsum("bihd,bjhd->bhij", Q, K) * 0.125
    p = jax.nn.softmax(s, axis=-1)
    attn = jnp.einsum("bhij,bjhd->bihd", p, V).reshape(ROWS, HQ_SHARD * DH)

    out = _flat16_allreduce(attn, Wo)
    return out.reshape(B, SQ, D)


# device time: 24813 ns/iter; 1.2032x vs baseline; 1.2022x over previous
import jax
import jax.numpy as jnp
from jax import lax
from jax.experimental import pallas as pl
from jax.experimental.pallas import tpu as pltpu

N_DEV = 32
B, SQ, D = 2, 128, 512
HQ_SHARD = 8
GQA = 4
DH = 64
ROWS = B * SQ
CHUNK = ROWS // N_DEV


def _allreduce_body(p_ref, out_ref, sendbuf, recvbuf, send_sems, recv_sems,
                    credit_sem):
    me = lax.axis_index("i")
    left = lax.rem(me + N_DEV - 1, N_DEV)
    right = lax.rem(me + 1, N_DEV)

    barrier = pltpu.get_barrier_semaphore()
    for nbr in (left, right):
        pl.semaphore_signal(barrier, inc=1, device_id=(nbr,),
                            device_id_type=pl.DeviceIdType.MESH)
    pl.semaphore_wait(barrier, 2)

    out_ref[...] = p_ref[...]

    total_steps = 2 * (N_DEV - 1)
    for h in range(total_steps):
        slot = h % 2
        rs_phase = h < N_DEV - 1
        if rs_phase:
            c_send = lax.rem(me - h + 2 * N_DEV, N_DEV)
            c_recv = lax.rem(me - h - 1 + 2 * N_DEV, N_DEV)
        else:
            g = h - (N_DEV - 1)
            c_send = lax.rem(me + 1 - g + 2 * N_DEV, N_DEV)
            c_recv = lax.rem(me - g + 2 * N_DEV, N_DEV)

        sendbuf[slot] = out_ref[pl.ds(c_send * CHUNK, CHUNK), :]

        if h >= 2:
            pl.semaphore_wait(credit_sem, 1)

        rdma = pltpu.make_async_remote_copy(
            src_ref=sendbuf.at[slot],
            dst_ref=recvbuf.at[slot],
            send_sem=send_sems.at[slot],
            recv_sem=recv_sems.at[slot],
            device_id=(right,),
            device_id_type=pl.DeviceIdType.MESH,
        )
        rdma.start()
        rdma.wait()

        if rs_phase:
            cur = out_ref[pl.ds(c_recv * CHUNK, CHUNK), :]
            out_ref[pl.ds(c_recv * CHUNK, CHUNK), :] = cur + recvbuf[slot]
        else:
            out_ref[pl.ds(c_recv * CHUNK, CHUNK), :] = recvbuf[slot]

        if h < total_steps - 2:
            pl.semaphore_signal(credit_sem, inc=1, device_id=(left,),
                                device_id_type=pl.DeviceIdType.MESH)


_PLANE = 8
_NZ = 4
_BLK1 = ROWS // _PLANE
_BLK2 = _BLK1 // _NZ


def _entry_barrier(z, q):
    barrier = pltpu.get_barrier_semaphore()
    n_peers = 0
    for dq in range(1, _PLANE):
        tgt = z * _PLANE + lax.rem(q + dq, _PLANE)
        pl.semaphore_signal(barrier, inc=1, device_id=(tgt,),
                            device_id_type=pl.DeviceIdType.MESH)
        n_peers += 1
    for dr in range(1, _NZ):
        tgt = lax.rem(z + dr, _NZ) * _PLANE + q
        pl.semaphore_signal(barrier, inc=1, device_id=(tgt,),
                            device_id_type=pl.DeviceIdType.MESH)
        n_peers += 1
    pl.semaphore_wait(barrier, n_peers)
    return n_peers


def _hier_body(p_ref, out_ref, rbuf1, rbuf2, send_sems,
               r_sems1, r_sems2, r_sems3, r_sems4):
    me = lax.axis_index("i")
    z = lax.div(me, _PLANE)
    q = lax.rem(me, _PLANE)

    n_peers = _entry_barrier(z, q)

    out_ref[...] = p_ref[...]
    _hier_collective(me, z, q, n_peers, out_ref, rbuf1, rbuf2, send_sems,
                     r_sems1, r_sems2, r_sems3, r_sems4)


def _hier_collective(me, z, q, n_peers, out_ref, rbuf1, rbuf2, send_sems,
                     r_sems1, r_sems2, r_sems3, r_sems4):
    my_blk = pl.multiple_of(q * _BLK1, 8)
    my_chunk = pl.multiple_of(q * _BLK1 + z * _BLK2, 8)

    sends = []
    for dq in range(1, _PLANE):
        qq = lax.rem(q + dq, _PLANE)
        tgt = z * _PLANE + qq
        r = pltpu.make_async_remote_copy(
            src_ref=out_ref.at[pl.ds(pl.multiple_of(qq * _BLK1, 8), _BLK1), :],
            dst_ref=rbuf1.at[pl.ds(my_blk, _BLK1), :],
            send_sem=send_sems.at[dq],
            recv_sem=r_sems1.at[_PLANE - dq],
            device_id=(tgt,),
            device_id_type=pl.DeviceIdType.MESH,
        )
        r.start()
        sends.append(r)
    acc = out_ref[pl.ds(my_blk, _BLK1), :]
    for dq in range(1, _PLANE):
        qq = lax.rem(q + dq, _PLANE)
        w = pltpu.make_async_remote_copy(
            src_ref=rbuf1.at[pl.ds(pl.multiple_of(qq * _BLK1, 8), _BLK1), :],
            dst_ref=rbuf1.at[pl.ds(pl.multiple_of(qq * _BLK1, 8), _BLK1), :],
            send_sem=send_sems.at[dq],
            recv_sem=r_sems1.at[dq],
            device_id=(me,),
            device_id_type=pl.DeviceIdType.MESH,
        )
        w.wait_recv()
        acc = acc + rbuf1[pl.ds(pl.multiple_of(qq * _BLK1, 8), _BLK1), :]
    out_ref[pl.ds(my_blk, _BLK1), :] = acc
    for r in sends:
        r.wait_send()

    sends = []
    for dr in range(1, _NZ):
        rr = lax.rem(z + dr, _NZ)
        tgt = rr * _PLANE + q
        src_off = pl.multiple_of(q * _BLK1 + rr * _BLK2, 8)
        r = pltpu.make_async_remote_copy(
            src_ref=out_ref.at[pl.ds(src_off, _BLK2), :],
            dst_ref=rbuf2.at[pl.ds(pl.multiple_of(z * _BLK2, 8), _BLK2), :],
            send_sem=send_sems.at[dr],
            recv_sem=r_sems2.at[_NZ - dr],
            device_id=(tgt,),
            device_id_type=pl.DeviceIdType.MESH,
        )
        r.start()
        sends.append(r)
    acc = out_ref[pl.ds(my_chunk, _BLK2), :]
    for dr in range(1, _NZ):
        rr = lax.rem(z + dr, _NZ)
        w = pltpu.make_async_remote_copy(
            src_ref=rbuf2.at[pl.ds(pl.multiple_of(rr * _BLK2, 8), _BLK2), :],
            dst_ref=rbuf2.at[pl.ds(pl.multiple_of(rr * _BLK2, 8), _BLK2), :],
            send_sem=send_sems.at[dr],
            recv_sem=r_sems2.at[dr],
            device_id=(me,),
            device_id_type=pl.DeviceIdType.MESH,
        )
        w.wait_recv()
        acc = acc + rbuf2[pl.ds(pl.multiple_of(rr * _BLK2, 8), _BLK2), :]
    out_ref[pl.ds(my_chunk, _BLK2), :] = acc
    for r in sends:
        r.wait_send()

    sends = []
    for dr in range(1, _NZ):
        rr = lax.rem(z + dr, _NZ)
        tgt = rr * _PLANE + q
        r = pltpu.make_async_remote_copy(
            src_ref=out_ref.at[pl.ds(my_chunk, _BLK2), :],
            dst_ref=out_ref.at[pl.ds(my_chunk, _BLK2), :],
            send_sem=send_sems.at[dr],
            recv_sem=r_sems3.at[_NZ - dr],
            device_id=(tgt,),
            device_id_type=pl.DeviceIdType.MESH,
        )
        r.start()
        sends.append(r)
    for dr in range(1, _NZ):
        rr = lax.rem(z + dr, _NZ)
        their = pl.multiple_of(q * _BLK1 + rr * _BLK2, 8)
        w = pltpu.make_async_remote_copy(
            src_ref=out_ref.at[pl.ds(their, _BLK2), :],
            dst_ref=out_ref.at[pl.ds(their, _BLK2), :],
            send_sem=send_sems.at[dr],
            recv_sem=r_sems3.at[dr],
            device_id=(me,),
            device_id_type=pl.DeviceIdType.MESH,
        )
        w.wait_recv()
    for r in sends:
        r.wait_send()

    sends = []
    for dq in range(1, _PLANE):
        qq = lax.rem(q + dq, _PLANE)
        tgt = z * _PLANE + qq
        r = pltpu.make_async_remote_copy(
            src_ref=out_ref.at[pl.ds(my_blk, _BLK1), :],
            dst_ref=out_ref.at[pl.ds(my_blk, _BLK1), :],
            send_sem=send_sems.at[dq],
            recv_sem=r_sems4.at[_PLANE - dq],
            device_id=(tgt,),
            device_id_type=pl.DeviceIdType.MESH,
        )
        r.start()
        sends.append(r)
    for dq in range(1, _PLANE):
        qq = lax.rem(q + dq, _PLANE)
        their = pl.multiple_of(qq * _BLK1, 8)
        w = pltpu.make_async_remote_copy(
            src_ref=out_ref.at[pl.ds(their, _BLK1), :],
            dst_ref=out_ref.at[pl.ds(their, _BLK1), :],
            send_sem=send_sems.at[dq],
            recv_sem=r_sems4.at[dq],
            device_id=(me,),
            device_id_type=pl.DeviceIdType.MESH,
        )
        w.wait_recv()
    for r in sends:
        r.wait_send()

    import functools

    @functools.partial(pl.run_scoped, exit_sem=pltpu.SemaphoreType.REGULAR)
    def _(exit_sem):
        for dq in range(1, _PLANE):
            tgt = z * _PLANE + lax.rem(q + dq, _PLANE)
            pl.semaphore_signal(exit_sem, inc=1, device_id=(tgt,),
                                device_id_type=pl.DeviceIdType.MESH)
        for dr in range(1, _NZ):
            tgt = lax.rem(z + dr, _NZ) * _PLANE + q
            pl.semaphore_signal(exit_sem, inc=1, device_id=(tgt,),
                                device_id_type=pl.DeviceIdType.MESH)
        pl.semaphore_wait(exit_sem, n_peers)


def _flat_body(p_ref, out_ref, rbuf, send_sems, rs_sems, ag_sems):
    me = lax.axis_index("i")

    import os as _os2
    if _os2.environ.get("DBG_PROBE", "") == "minbar":
        with jax.named_scope("entry_barrier"):
            barrier = pltpu.get_barrier_semaphore()
            pl.semaphore_signal(barrier, inc=1,
                                device_id=(lax.rem(me + 1, N_DEV),),
                                device_id_type=pl.DeviceIdType.MESH)
            pl.semaphore_wait(barrier, 1)
    else:
        with jax.named_scope("entry_barrier"):
            barrier = pltpu.get_barrier_semaphore()
            for d in range(1, N_DEV):
                tgt = lax.rem(me + d, N_DEV)
                pl.semaphore_signal(barrier, inc=1, device_id=(tgt,),
                                    device_id_type=pl.DeviceIdType.MESH)
            pl.semaphore_wait(barrier, N_DEV - 1)

    out_ref[...] = p_ref[...]
    my_off = pl.multiple_of(me * CHUNK, 8)

    import os as _os
    _PROBE = _os.environ.get("DBG_PROBE", "")
    if _PROBE in ("barrier", "minbar"):
        return

    sends = []
    with jax.named_scope("rs_issue"):
        for d in range(1, N_DEV):
            tgt = lax.rem(me + d, N_DEV)
            r = pltpu.make_async_remote_copy(
                src_ref=out_ref.at[pl.ds(pl.multiple_of(tgt * CHUNK, 8), CHUNK), :],
                dst_ref=rbuf.at[pl.ds(my_off, CHUNK), :],
                send_sem=send_sems.at[d],
                recv_sem=rs_sems.at[me],
                device_id=(tgt,),
                device_id_type=pl.DeviceIdType.MESH,
            )
            r.start()
            sends.append(r)
        rbuf[pl.ds(my_off, CHUNK), :] = out_ref[pl.ds(my_off, CHUNK), :]
    with jax.named_scope("rs_wait"):
        for d in range(1, N_DEV):
            src = lax.rem(me + d, N_DEV)
            w = pltpu.make_async_remote_copy(
                src_ref=rbuf.at[pl.ds(pl.multiple_of(src * CHUNK, 8), CHUNK), :],
                dst_ref=rbuf.at[pl.ds(pl.multiple_of(src * CHUNK, 8), CHUNK), :],
                send_sem=send_sems.at[d],
                recv_sem=rs_sems.at[src],
                device_id=(me,),
                device_id_type=pl.DeviceIdType.MESH,
            )
            w.wait_recv()
    with jax.named_scope("rs_tree_sum"):
        for half in (128, 64, 32, 16, 8):
            a = rbuf[0:half, :]
            b = rbuf[half:2 * half, :]
            rbuf[0:half, :] = a + b
        out_ref[pl.ds(my_off, CHUNK), :] = rbuf[0:CHUNK, :]
        for r in sends:
            r.wait_send()

    if _PROBE == "rs":
        return

    sends = []
    with jax.named_scope("ag_issue"):
        for d in range(1, N_DEV):
            tgt = lax.rem(me + d, N_DEV)
            r = pltpu.make_async_remote_copy(
                src_ref=out_ref.at[pl.ds(my_off, CHUNK), :],
                dst_ref=out_ref.at[pl.ds(my_off, CHUNK), :],
                send_sem=send_sems.at[d],
                recv_sem=ag_sems.at[me],
                device_id=(tgt,),
                device_id_type=pl.DeviceIdType.MESH,
            )
            r.start()
            sends.append(r)
    with jax.named_scope("ag_wait"):
        for d in range(1, N_DEV):
            src = lax.rem(me + d, N_DEV)
            their = pl.multiple_of(src * CHUNK, 8)
            w = pltpu.make_async_remote_copy(
                src_ref=out_ref.at[pl.ds(their, CHUNK), :],
                dst_ref=out_ref.at[pl.ds(their, CHUNK), :],
                send_sem=send_sems.at[d],
                recv_sem=ag_sems.at[src],
                device_id=(me,),
                device_id_type=pl.DeviceIdType.MESH,
            )
            w.wait_recv()
        for r in sends:
            r.wait_send()


def _flat_allreduce(partial):
    return pl.pallas_call(
        _flat_body,
        out_shape=jax.ShapeDtypeStruct((ROWS, D), jnp.float32),
        in_specs=[pl.BlockSpec(memory_space=pltpu.VMEM)],
        out_specs=pl.BlockSpec(memory_space=pltpu.VMEM),
        scratch_shapes=[
            pltpu.VMEM((ROWS, D), jnp.float32),
            pltpu.SemaphoreType.DMA((N_DEV,)),
            pltpu.SemaphoreType.DMA((N_DEV,)),
            pltpu.SemaphoreType.DMA((N_DEV,)),
        ],
        compiler_params=pltpu.CompilerParams(collective_id=0),
    )(partial)


_C_R, _C_L = 16, 256
_N_RB = ROWS // _C_R
_BF = jnp.bfloat16


def _chunk_offs(c):
    return (pl.multiple_of(lax.rem(c, _N_RB) * _C_R, 16),
            pl.multiple_of(lax.div(c, _N_RB) * _C_L, 128))


def _flat16_body(p_ref, out_ref, rbuf, agbuf,
                 send_sems, rs_sems, ag_sems):
    me = lax.axis_index("i")

    barrier = pltpu.get_barrier_semaphore()
    for d in range(1, N_DEV):
        tgt = lax.rem(me + d, N_DEV)
        pl.semaphore_signal(barrier, inc=1, device_id=(tgt,),
                            device_id_type=pl.DeviceIdType.MESH)
    pl.semaphore_wait(barrier, N_DEV - 1)

    sends = []
    for d in range(1, N_DEV):
        tgt = lax.rem(me + d, N_DEV)
        ro, lo = _chunk_offs(tgt)
        r = pltpu.make_async_remote_copy(
            src_ref=p_ref.at[pl.ds(ro, _C_R), pl.ds(lo, _C_L)],
            dst_ref=rbuf.at[me],
            send_sem=send_sems.at[d],
            recv_sem=rs_sems.at[me],
            device_id=(tgt,),
            device_id_type=pl.DeviceIdType.MESH,
        )
        r.start()
        sends.append(r)
    my_ro, my_lo = _chunk_offs(me)
    rbuf[me] = p_ref[pl.ds(my_ro, _C_R), pl.ds(my_lo, _C_L)]
    for d in range(1, N_DEV):
        src = lax.rem(me + d, N_DEV)
        w = pltpu.make_async_remote_copy(
            src_ref=rbuf.at[src],
            dst_ref=rbuf.at[src],
            send_sem=send_sems.at[d],
            recv_sem=rs_sems.at[src],
            device_id=(me,),
            device_id_type=pl.DeviceIdType.MESH,
        )
        w.wait_recv()
    for half in (16, 8, 4, 2, 1):
        rbuf[0:half] = rbuf[0:half] + rbuf[half:2 * half]
    for r in sends:
        r.wait_send()

    sends = []
    for d in range(1, N_DEV):
        tgt = lax.rem(me + d, N_DEV)
        r = pltpu.make_async_remote_copy(
            src_ref=rbuf.at[0],
            dst_ref=agbuf.at[me],
            send_sem=send_sems.at[d],
            recv_sem=ag_sems.at[me],
            device_id=(tgt,),
            device_id_type=pl.DeviceIdType.MESH,
        )
        r.start()
        sends.append(r)
    agbuf[me] = rbuf[0]
    for d in range(1, N_DEV):
        src = lax.rem(me + d, N_DEV)
        w = pltpu.make_async_remote_copy(
            src_ref=agbuf.at[src],
            dst_ref=agbuf.at[src],
            send_sem=send_sems.at[d],
            recv_sem=ag_sems.at[src],
            device_id=(me,),
            device_id_type=pl.DeviceIdType.MESH,
        )
        w.wait_recv()
    for c in range(N_DEV):
        ro = (c % _N_RB) * _C_R
        lo = (c // _N_RB) * _C_L
        out_ref[ro:ro + _C_R, lo:lo + _C_L] = agbuf[c].astype(jnp.float32)
    for r in sends:
        r.wait_send()


def _flat16_allreduce(partial):
    return pl.pallas_call(
        _flat16_body,
        out_shape=jax.ShapeDtypeStruct((ROWS, D), jnp.float32),
        in_specs=[pl.BlockSpec(memory_space=pltpu.VMEM)],
        out_specs=pl.BlockSpec(memory_space=pltpu.VMEM),
        scratch_shapes=[
            pltpu.VMEM((N_DEV, _C_R, _C_L), _BF),
            pltpu.VMEM((N_DEV, _C_R, _C_L), _BF),
            pltpu.SemaphoreType.DMA((N_DEV,)),
            pltpu.SemaphoreType.DMA((N_DEV,)),
            pltpu.SemaphoreType.DMA((N_DEV,)),
        ],
        compiler_params=pltpu.CompilerParams(collective_id=0),
    )(partial)


def _fused_body(x_ref, wq_ref, wo_ref, wk_ref, wv_ref, out_ref,
                attn_ref, rbuf1, rbuf2, send_sems,
                r_sems1, r_sems2, r_sems3, r_sems4):
    me = lax.axis_index("i")
    z = lax.div(me, _PLANE)
    q = lax.rem(me, _PLANE)

    n_peers = _entry_barrier(z, q)

    x2 = x_ref[...]
    Q = jnp.dot(x2, wq_ref[...], preferred_element_type=jnp.float32)
    kv_col = pl.multiple_of(me * (HQ_SHARD // GQA) * DH, 128)
    K2 = jnp.dot(x2, wk_ref[:, pl.ds(kv_col, (HQ_SHARD // GQA) * DH)],
                 preferred_element_type=jnp.float32)
    V2 = jnp.dot(x2, wv_ref[:, pl.ds(kv_col, (HQ_SHARD // GQA) * DH)],
                 preferred_element_type=jnp.float32)

    for b in range(B):
        rows = slice(b * SQ, (b + 1) * SQ)
        for h in range(HQ_SHARD):
            qh = Q[rows, h * DH:(h + 1) * DH]
            kh = K2[rows, (h // GQA) * DH:(h // GQA + 1) * DH]
            vh = V2[rows, (h // GQA) * DH:(h // GQA + 1) * DH]
            s = lax.dot_general(qh, kh, (((1,), (1,)), ((), ())),
                                preferred_element_type=jnp.float32) * 0.125
            m = jnp.max(s, axis=1, keepdims=True)
            p = jnp.exp(s - m)
            p = p / jnp.sum(p, axis=1, keepdims=True)
            attn_ref[rows, h * DH:(h + 1) * DH] = jnp.dot(
                p, vh, preferred_element_type=jnp.float32)

    out_ref[...] = jnp.dot(attn_ref[...], wo_ref[...],
                           preferred_element_type=jnp.float32)

    _hier_collective(me, z, q, n_peers, out_ref, rbuf1, rbuf2, send_sems,
                     r_sems1, r_sems2, r_sems3, r_sems4)


def _fused_attn_allreduce(x2, Wq, Wo, Wk, Wv):
    return pl.pallas_call(
        _fused_body,
        out_shape=jax.ShapeDtypeStruct((ROWS, D), jnp.float32),
        in_specs=[pl.BlockSpec(memory_space=pltpu.VMEM)] * 5,
        out_specs=pl.BlockSpec(memory_space=pltpu.VMEM),
        scratch_shapes=[
            pltpu.VMEM((ROWS, HQ_SHARD * DH), jnp.float32),
            pltpu.VMEM((ROWS, D), jnp.float32),
            pltpu.VMEM((_BLK1, D), jnp.float32),
            pltpu.SemaphoreType.DMA((_PLANE,)),
            pltpu.SemaphoreType.DMA((_PLANE,)),
            pltpu.SemaphoreType.DMA((_NZ,)),
            pltpu.SemaphoreType.DMA((_NZ,)),
            pltpu.SemaphoreType.DMA((_PLANE,)),
        ],
        compiler_params=pltpu.CompilerParams(collective_id=0),
    )(x2, Wq, Wo, Wk, Wv)


def _hier_allreduce(partial):
    return pl.pallas_call(
        _hier_body,
        out_shape=jax.ShapeDtypeStruct((ROWS, D), jnp.float32),
        in_specs=[pl.BlockSpec(memory_space=pltpu.VMEM)],
        out_specs=pl.BlockSpec(memory_space=pltpu.VMEM),
        scratch_shapes=[
            pltpu.VMEM((ROWS, D), jnp.float32),
            pltpu.VMEM((_BLK1, D), jnp.float32),
            pltpu.SemaphoreType.DMA((_PLANE,)),
            pltpu.SemaphoreType.DMA((_PLANE,)),
            pltpu.SemaphoreType.DMA((_NZ,)),
            pltpu.SemaphoreType.DMA((_NZ,)),
            pltpu.SemaphoreType.DMA((_PLANE,)),
        ],
        compiler_params=pltpu.CompilerParams(collective_id=0),
    )(partial)


_RS_MASKS = (16, 8, 4, 2, 1)
_AG_MASKS = (1, 2, 4, 8, 16)
_RS_SIZES = (128, 64, 32, 16, 8)
_AG_SIZES = (8, 16, 32, 64, 128)
_RS_OFF = (0, 128, 192, 224, 240)
_AG_OFF = (248, 256, 272, 304, 368)
_RECV_ROWS = 496


def _butterfly_body(p_ref, out_ref, recvbuf, send_sems, recv_sems):
    me = lax.axis_index("i")
    partners = [jnp.bitwise_xor(me, m) for m in (1, 2, 4, 8, 16)]

    barrier = pltpu.get_barrier_semaphore()
    for p in partners:
        pl.semaphore_signal(barrier, inc=1, device_id=(p,),
                            device_id_type=pl.DeviceIdType.MESH)
    pl.semaphore_wait(barrier, len(partners))

    out_ref[...] = p_ref[...]

    off = jnp.int32(0)
    for k, m in enumerate(_RS_MASKS):
        partner = jnp.bitwise_xor(me, m)
        half = _RS_SIZES[k]
        keep_low = jnp.bitwise_and(me, m) == 0
        send_off = pl.multiple_of(off + jnp.where(keep_low, half, 0), 8)
        keep_off = pl.multiple_of(off + jnp.where(keep_low, 0, half), 8)
        rdma = pltpu.make_async_remote_copy(
            src_ref=out_ref.at[pl.ds(send_off, half), :],
            dst_ref=recvbuf.at[pl.ds(_RS_OFF[k], half), :],
            send_sem=send_sems.at[k],
            recv_sem=recv_sems.at[k],
            device_id=(partner,),
            device_id_type=pl.DeviceIdType.MESH,
        )
        rdma.start()
        rdma.wait()
        cur = out_ref[pl.ds(keep_off, half), :]
        out_ref[pl.ds(keep_off, half), :] = (
            cur + recvbuf[pl.ds(_RS_OFF[k], half), :]
        )
        off = keep_off

    for j, m in enumerate(_AG_MASKS):
        k = 5 + j
        partner = jnp.bitwise_xor(me, m)
        sz = _AG_SIZES[j]
        their_off = pl.multiple_of(jnp.bitwise_xor(off, 8 * m), 8)
        rdma = pltpu.make_async_remote_copy(
            src_ref=out_ref.at[pl.ds(pl.multiple_of(off, 8), sz), :],
            dst_ref=recvbuf.at[pl.ds(_AG_OFF[j], sz), :],
            send_sem=send_sems.at[k],
            recv_sem=recv_sems.at[k],
            device_id=(partner,),
            device_id_type=pl.DeviceIdType.MESH,
        )
        rdma.start()
        rdma.wait()
        out_ref[pl.ds(their_off, sz), :] = recvbuf[pl.ds(_AG_OFF[j], sz), :]
        off = jnp.bitwise_and(off, jnp.int32(~(8 * m)))

    import functools

    @functools.partial(pl.run_scoped, exit_sem=pltpu.SemaphoreType.REGULAR)
    def _(exit_sem):
        for p in partners:
            pl.semaphore_signal(exit_sem, inc=1, device_id=(p,),
                                device_id_type=pl.DeviceIdType.MESH)
        pl.semaphore_wait(exit_sem, len(partners))


def _butterfly_allreduce(partial):
    return pl.pallas_call(
        _butterfly_body,
        out_shape=jax.ShapeDtypeStruct((ROWS, D), jnp.float32),
        in_specs=[pl.BlockSpec(memory_space=pltpu.VMEM)],
        out_specs=pl.BlockSpec(memory_space=pltpu.VMEM),
        scratch_shapes=[
            pltpu.VMEM((_RECV_ROWS, D), jnp.float32),
            pltpu.SemaphoreType.DMA((10,)),
            pltpu.SemaphoreType.DMA((10,)),
        ],
        compiler_params=pltpu.CompilerParams(collective_id=0),
    )(partial)


def _ring_allreduce(partial):
    return pl.pallas_call(
        _allreduce_body,
        out_shape=jax.ShapeDtypeStruct((ROWS, D), jnp.float32),
        in_specs=[pl.BlockSpec(memory_space=pltpu.VMEM)],
        out_specs=pl.BlockSpec(memory_space=pltpu.VMEM),
        scratch_shapes=[
            pltpu.VMEM((2, CHUNK, D), jnp.float32),
            pltpu.VMEM((2, CHUNK, D), jnp.float32),
            pltpu.SemaphoreType.DMA((2,)),
            pltpu.SemaphoreType.DMA((2,)),
            pltpu.SemaphoreType.REGULAR,
        ],
        compiler_params=pltpu.CompilerParams(collective_id=0),
    )(partial)


def kernel(x, Wq, Wo, Wk, Wv):
    i = lax.axis_index("i")
    x2 = x.reshape(ROWS, D)

    Q = (x2 @ Wq).reshape(B, SQ, HQ_SHARD, DH)

    kv_cols = (HQ_SHARD // GQA) * DH
    Wk_s = lax.dynamic_slice_in_dim(Wk, i * kv_cols, kv_cols, axis=1)
    Wv_s = lax.dynamic_slice_in_dim(Wv, i * kv_cols, kv_cols, axis=1)
    K = (x2 @ Wk_s).reshape(B, SQ, HQ_SHARD // GQA, DH)
    V = (x2 @ Wv_s).reshape(B, SQ, HQ_SHARD // GQA, DH)
    K = jnp.repeat(K, GQA, axis=2)
    V = jnp.repeat(V, GQA, axis=2)

    s = jnp.einsum("bihd,bjhd->bhij", Q, K) * 0.125
    p = jax.nn.softmax(s, axis=-1)
    attn = jnp.einsum("bhij,bjhd->bihd", p, V).reshape(ROWS, HQ_SHARD * DH)

    partial = (attn @ Wo).astype(jnp.bfloat16)

    out = _flat16_allreduce(partial)
    return out.reshape(B, SQ, D)
